# baseline (speedup 1.0000x reference)
import functools

import jax
import jax.numpy as jnp
from jax import lax
from jax.experimental import pallas as pl
from jax.experimental.pallas import tpu as pltpu
from jax.experimental.pallas import tpu_sc as plsc

N = 4096
T = 4
E = 32768
D = 16
H = 2
TE = T * E
NC = 2
NS = 16
L = 16
CE = TE // NS
RPC = N // NC
RPT = RPC // NS
CHUNK = 128
NCHUNK = CE // CHUNK
W1 = 16
W2 = 8


def _body(scores_hbm, ew_hbm, seg_hbm, ridx_hbm, par_hbm, out_hbm,
          scores_v, ew_v, seg2_v, ridx2_v,
          vals0_v, vals1_v, v0b_v, v1b_v, g0_v, gs0_v, g1_v, gs1_v,
          den0_v, num0_v, den1_v, num1_v, obuf_v, bt0_v, bt1_v, par_v,
          segval0_sh, segval1_sh, den0_sh, num0_sh, den1_sh, num1_sh,
          sem):
    c = lax.axis_index("c")
    s = lax.axis_index("s")

    pltpu.sync_copy(scores_hbm, scores_v)
    pltpu.sync_copy(ew_hbm.at[pl.ds(s * CE, CE)], ew_v)
    pltpu.sync_copy(seg_hbm.at[pl.ds(s * NCHUNK, NCHUNK)], seg2_v)
    rbase = c * (TE // CHUNK) + s * NCHUNK
    pltpu.sync_copy(ridx_hbm.at[pl.ds(rbase, NCHUNK)], ridx2_v)
    pltpu.sync_copy(par_hbm, par_v)

    def zbody(i, _):
        vals0_v[pl.ds(i * L, L)] = jnp.zeros((L,), jnp.float32)
        return _
    lax.fori_loop(0, CE // L, zbody, None)
    pltpu.sync_copy(vals0_v, segval0_sh.at[pl.ds(s * CE, CE)])
    pltpu.sync_copy(vals0_v, segval1_sh.at[pl.ds(s * CE, CE)])

    @pl.when(s == 0)
    def _zero_rows():
        zsl = vals0_v.at[pl.ds(0, RPC + 8)]
        pltpu.sync_copy(zsl, den0_sh)
        pltpu.sync_copy(zsl, num0_sh)
        pltpu.sync_copy(zsl, den1_sh)
        pltpu.sync_copy(zsl, num1_sh)

    lane = lax.iota(jnp.int32, L)

    def lanesum(x):
        for bit in (8, 4, 2, 1):
            obuf_v[pl.ds(0, L)] = x
            x = x + plsc.load_gather(obuf_v, [lane ^ bit])
        return x

    wm0 = par_v[4, :]
    wm1 = par_v[5, :]
    a0v = par_v[6, :]
    c0v = par_v[7, :]
    a1v = par_v[8, :]
    c1v = par_v[9, :]
    bt0 = jnp.zeros((L,), jnp.float32)
    bt1 = jnp.zeros((L,), jnp.float32)
    for t in range(T):
        etv = par_v[t, :]
        bt0 = jnp.where(lane == t, lanesum(etv * wm0), bt0)
        bt1 = jnp.where(lane == t, lanesum(etv * wm1), bt1)
    bt0_v[pl.ds(0, L)] = bt0
    bt1_v[pl.ds(0, L)] = bt1

    def sbody(i, acc):
        return acc + scores_v[pl.ds(i * L, L)]
    accv = lax.fori_loop(0, N // L, sbody, jnp.zeros((L,), jnp.float32))
    S = lanesum(accv)

    plsc.subcore_barrier()

    def p1(k, _):
        sl = pl.ds(k * L, L)
        w = ew_v[sl]
        sv = w & 0xFFF
        dv = (w >> 12) & 0xFFF
        tv = (w >> 24) & 0x3
        ss = plsc.load_gather(scores_v, [sv])
        sd = plsc.load_gather(scores_v, [dv])
        b0e = plsc.load_gather(bt0_v, [tv])
        b1e = plsc.load_gather(bt1_v, [tv])
        vals0_v[sl] = a0v * ss + c0v * sd + b0e
        vals1_v[sl] = a1v * ss + c1v * sd + b1e
        return _
    lax.fori_loop(0, CE // L, p1, None)

    def scat1(wv, _):
        descs = []
        for i in range(W1):
            j = wv * W1 + i
            idxrow = seg2_v.at[j]
            vsl = pl.ds(j * CHUNK, CHUNK)
            descs.append(pltpu.async_copy(
                vals0_v.at[vsl], segval0_sh.at[idxrow], sem, add=True))
            descs.append(pltpu.async_copy(
                vals1_v.at[vsl], segval1_sh.at[idxrow], sem, add=True))
        for dsc in descs:
            dsc.wait()
        return _
    lax.fori_loop(0, NCHUNK // W1, scat1, None)

    plsc.subcore_barrier()

    def gat2(wv, _):
        descs = []
        for i in range(W1):
            j = wv * W1 + i
            idxrow = seg2_v.at[j]
            vsl = pl.ds(j * CHUNK, CHUNK)
            descs.append(pltpu.async_copy(
                segval0_sh.at[idxrow], v0b_v.at[vsl], sem))
            descs.append(pltpu.async_copy(
                segval1_sh.at[idxrow], v1b_v.at[vsl], sem))
        for dsc in descs:
            dsc.wait()
        return _
    lax.fori_loop(0, NCHUNK // W1, gat2, None)

    def p2(k, _):
        sl = pl.ds(k * L, L)
        w = ew_v[sl]
        dv = (w >> 12) & 0xFFF
        lastf = ((w >> 26) & 0x1).astype(jnp.float32)
        sd = plsc.load_gather(scores_v, [dv])
        v0 = v0b_v[sl]
        v1 = v1b_v[sl]
        e0 = (jnp.exp(jnp.where(v0 >= 0, v0, 0.2 * v0)) - 1.0) * lastf
        e1 = (jnp.exp(jnp.where(v1 >= 0, v1, 0.2 * v1)) - 1.0) * lastf
        g0_v[sl] = e0
        gs0_v[sl] = e0 * sd
        g1_v[sl] = e1
        gs1_v[sl] = e1 * sd
        return _
    lax.fori_loop(0, CE // L, p2, None)

    def scat2(wv, _):
        descs = []
        for i in range(W2):
            j = wv * W2 + i
            idxrow = ridx2_v.at[j]
            vsl = pl.ds(j * CHUNK, CHUNK)
            descs.append(pltpu.async_copy(
                g0_v.at[vsl], den0_sh.at[idxrow], sem, add=True))
            descs.append(pltpu.async_copy(
                gs0_v.at[vsl], num0_sh.at[idxrow], sem, add=True))
            descs.append(pltpu.async_copy(
                g1_v.at[vsl], den1_sh.at[idxrow], sem, add=True))
            descs.append(pltpu.async_copy(
                gs1_v.at[vsl], num1_sh.at[idxrow], sem, add=True))
        for dsc in descs:
            dsc.wait()
        return _
    lax.fori_loop(0, NCHUNK // W2, scat2, None)

    plsc.subcore_barrier()

    rb = s * RPT
    pltpu.sync_copy(den0_sh.at[pl.ds(rb, RPT)], den0_v)
    pltpu.sync_copy(num0_sh.at[pl.ds(rb, RPT)], num0_v)
    pltpu.sync_copy(den1_sh.at[pl.ds(rb, RPT)], den1_v)
    pltpu.sync_copy(num1_sh.at[pl.ds(rb, RPT)], num1_v)
    fN = jnp.float32(N)

    def p3(k, _):
        sl = pl.ds(k * L, L)
        o = 0.5 * ((S + num0_v[sl]) / (fN + den0_v[sl])
                   + (S + num1_v[sl]) / (fN + den1_v[sl]))
        obuf_v[sl] = o
        return _
    lax.fori_loop(0, RPT // L, p3, None)
    pltpu.sync_copy(obuf_v, out_hbm.at[pl.ds(c * RPC + rb, RPT)])


_sc_call = functools.partial(
    pl.kernel,
    out_type=jax.ShapeDtypeStruct((N,), jnp.float32),
    mesh=plsc.VectorSubcoreMesh(core_axis_name="c", subcore_axis_name="s"),
    compiler_params=pltpu.CompilerParams(needs_layout_passes=False),
    scratch_types=[
        pltpu.VMEM((N,), jnp.float32),
        pltpu.VMEM((CE,), jnp.int32),
        pltpu.VMEM((NCHUNK, CHUNK), jnp.int32),
        pltpu.VMEM((NCHUNK, CHUNK), jnp.int32),
        pltpu.VMEM((CE,), jnp.float32),
        pltpu.VMEM((CE,), jnp.float32),
        pltpu.VMEM((CE,), jnp.float32),
        pltpu.VMEM((CE,), jnp.float32),
        pltpu.VMEM((CE,), jnp.float32),
        pltpu.VMEM((CE,), jnp.float32),
        pltpu.VMEM((CE,), jnp.float32),
        pltpu.VMEM((CE,), jnp.float32),
        pltpu.VMEM((RPT,), jnp.float32),
        pltpu.VMEM((RPT,), jnp.float32),
        pltpu.VMEM((RPT,), jnp.float32),
        pltpu.VMEM((RPT,), jnp.float32),
        pltpu.VMEM((RPT,), jnp.float32),
        pltpu.VMEM((L,), jnp.float32),
        pltpu.VMEM((L,), jnp.float32),
        pltpu.VMEM((16, L), jnp.float32),
        pltpu.VMEM_SHARED((TE,), jnp.float32),
        pltpu.VMEM_SHARED((TE,), jnp.float32),
        pltpu.VMEM_SHARED((RPC + 8,), jnp.float32),
        pltpu.VMEM_SHARED((RPC + 8,), jnp.float32),
        pltpu.VMEM_SHARED((RPC + 8,), jnp.float32),
        pltpu.VMEM_SHARED((RPC + 8,), jnp.float32),
        pltpu.SemaphoreType.DMA,
    ],
)(_body)


def kernel(scores, edge_index, edge_type_emb, attention_weight):
    scores1 = scores[:, 0]
    src = edge_index[:, 0, :].reshape(-1)
    dst = edge_index[:, 1, :].reshape(-1)

    key = src * N + dst
    order = jnp.argsort(key)
    src_s = src[order]
    dst_s = dst[order]
    sk = key[order]
    t_s = (order >> 15).astype(jnp.int32)
    is_new = jnp.concatenate(
        [jnp.ones((1,), jnp.int32), (sk[1:] != sk[:-1]).astype(jnp.int32)])
    seg = jnp.cumsum(is_new) - 1
    is_last = jnp.concatenate(
        [(sk[:-1] != sk[1:]).astype(jnp.int32), jnp.ones((1,), jnp.int32)])
    ew = src_s | (dst_s << 12) | (t_s << 24) | (is_last << 26)
    r0 = jnp.where(src_s < RPC, src_s, RPC)
    r1 = jnp.where(src_s >= RPC, src_s - RPC, RPC)
    ridx2 = jnp.concatenate([r0, r1]).reshape(2 * TE // CHUNK, CHUNK)
    seg2d = seg.reshape(TE // CHUNK, CHUNK)

    aw = attention_weight[:, :, 0]
    params = jnp.concatenate([
        edge_type_emb,
        aw[:, 1:D + 1],
        jnp.broadcast_to(aw[0, 0], (1, D)),
        jnp.broadcast_to(aw[0, D + 1], (1, D)),
        jnp.broadcast_to(aw[1, 0], (1, D)),
        jnp.broadcast_to(aw[1, D + 1], (1, D)),
        jnp.zeros((6, D), jnp.float32),
    ], axis=0)

    out = _sc_call(scores1, ew, seg2d, ridx2, params)
    return out.reshape(N, 1)
